# Initial kernel scaffold; baseline (speedup 1.0000x reference)
import jax
import jax.numpy as jnp
from jax.experimental import pallas as pl


def kernel(x, kernel, selection, k):
    raise NotImplementedError("write your pallas kernel here")


# speedup vs baseline: 1.0452x; 1.0452x over previous
import jax
import jax.numpy as jnp
from jax import lax
from jax.experimental import pallas as pl
from jax.experimental.pallas import tpu as pltpu

_D = 2048
_BR = 1024


def _body(sel_ref, k_ref, kvec_ref, x_ref, out_ref, kk_ref):
    @pl.when(pl.program_id(0) == 0)
    def _prologue():
        kv = kvec_ref[...]
        b = lax.bitcast_convert_type(kv, jnp.int32)
        u = lax.bitcast_convert_type(kv, jnp.uint32)
        key = jnp.where(b < 0, ~u, u | jnp.uint32(0x80000000))
        k = k_ref[0]

        def step(i, acc):
            bit = jnp.uint32(1) << (jnp.uint32(31) - i.astype(jnp.uint32))
            cand = acc | bit
            cnt = jnp.sum((key >= cand).astype(jnp.int32))
            return jnp.where(cnt >= k, cand, acc)

        thresh = lax.fori_loop(0, 32, step, jnp.uint32(0))
        masked = jnp.where(key < thresh, jnp.float32(0.0), kv)
        kk_ref[...] = jnp.where(sel_ref[0] != 0, masked, kv)

    out_ref[...] = x_ref[...] * kk_ref[...]


def kernel(x, kernel, selection, k):
    n_rows = x.shape[0]
    grid = (n_rows // _BR,)
    sel_arr = jnp.asarray(selection, jnp.int32).reshape(1)
    k_arr = jnp.asarray(k, jnp.int32).reshape(1)
    kvec = kernel.reshape(1, _D)

    return pl.pallas_call(
        _body,
        grid_spec=pltpu.PrefetchScalarGridSpec(
            num_scalar_prefetch=2,
            grid=grid,
            in_specs=[
                pl.BlockSpec((1, _D), lambda i, *_: (0, 0)),
                pl.BlockSpec((_BR, _D), lambda i, *_: (i, 0)),
            ],
            out_specs=pl.BlockSpec((_BR, _D), lambda i, *_: (i, 0)),
            scratch_shapes=[pltpu.VMEM((1, _D), jnp.float32)],
        ),
        out_shape=jax.ShapeDtypeStruct(x.shape, x.dtype),
    )(sel_arr, k_arr, kvec, x)
